# baseline (speedup 1.0000x reference)
import dataclasses
import functools

import jax
import jax.numpy as jnp
from jax import lax
from jax.experimental import pallas as pl
from jax.experimental.pallas import tpu as pltpu
from jax.experimental.pallas import tpu_sc as plsc

_LANES = 16
_WINDOW = 128
_NSLOT = 2


def _dense_body(x_ref, w_ref, b_ref, o_ref):
    o_ref[...] = (
        jnp.dot(x_ref[...], w_ref[...], preferred_element_type=jnp.float32)
        + b_ref[...]
    ).astype(jnp.bfloat16)


def _dense_bf16(x, W, b):
    n, d_in = x.shape
    d_out = W.shape[1]
    blk = 2000
    return pl.pallas_call(
        _dense_body,
        grid=(n // blk,),
        in_specs=[
            pl.BlockSpec((blk, d_in), lambda i: (i, 0)),
            pl.BlockSpec((d_in, d_out), lambda i: (0, 0)),
            pl.BlockSpec((1, d_out), lambda i: (0, 0)),
        ],
        out_specs=pl.BlockSpec((blk, d_out), lambda i: (i, 0)),
        out_shape=jax.ShapeDtypeStruct((n, d_out), jnp.bfloat16),
    )(x, W, b.reshape(1, d_out))


def _edge_combine(z_packed, idx_combined, e_total, d):
    nwin = e_total // _WINDOW
    n_nodes = z_packed.shape[0]
    dw = d // 2
    mesh = plsc.VectorSubcoreMesh(
        core_axis_name="core", subcore_axis_name="subcore"
    )
    n_workers = 32
    max_j = -(-nwin // n_workers)
    if max_j % _NSLOT:
        max_j += _NSLOT - max_j % _NSLOT

    cp = pltpu.CompilerParams()
    if "needs_layout_passes" in pltpu.CompilerParams.__dataclass_fields__:
        cp = dataclasses.replace(cp, needs_layout_passes=False)
    if "use_tc_tiling_on_sc" in pltpu.CompilerParams.__dataclass_fields__:
        cp = dataclasses.replace(cp, use_tc_tiling_on_sc=False)

    slot_types = [
        pltpu.VMEM((3 * _WINDOW,), jnp.int32),
        pltpu.VMEM((_WINDOW, dw), jnp.int32),
        pltpu.VMEM((_WINDOW, dw), jnp.int32),
        pltpu.VMEM((_WINDOW, d), jnp.float32),
        pltpu.VMEM((_WINDOW,), jnp.float32),
        pltpu.SemaphoreType.DMA,
        pltpu.SemaphoreType.DMA,
        pltpu.SemaphoreType.DMA,
    ]

    @functools.partial(
        pl.kernel,
        out_type=jax.ShapeDtypeStruct((e_total, d), jnp.float32),
        mesh=mesh,
        compiler_params=cp,
        scratch_types=slot_types * _NSLOT
        + [pltpu.VMEM_SHARED((n_nodes, dw), jnp.int32)],
    )
    def k(z_hbm, i_hbm, o_hbm, *scratch):
        zs = scratch[-1]
        ns = len(slot_types)
        slots = tuple(
            dict(
                ib=scratch[i * ns + 0],
                rb=scratch[i * ns + 1],
                cb=scratch[i * ns + 2],
                ob=scratch[i * ns + 3],
                vb=scratch[i * ns + 4],
                sem_i=scratch[i * ns + 5],
                sem_g=scratch[i * ns + 6],
                sem_o=scratch[i * ns + 7],
            )
            for i in range(_NSLOT)
        )
        t = lax.axis_index("subcore") * 2 + lax.axis_index("core")

        def win(j):
            return t + n_workers * j

        def issue_idx(j, s):
            @pl.when(win(j) < nwin)
            def _():
                pltpu.async_copy(i_hbm.at[win(j)], s["ib"], s["sem_i"])

        def issue_gather(j, s):
            @pl.when(win(j) < nwin)
            def _():
                pltpu.make_async_copy(i_hbm.at[0], s["ib"], s["sem_i"]).wait()
                pltpu.async_copy(
                    zs.at[s["ib"].at[pl.ds(0, _WINDOW)]], s["rb"], s["sem_g"]
                )
                pltpu.async_copy(
                    zs.at[s["ib"].at[pl.ds(_WINDOW, _WINDOW)]],
                    s["cb"],
                    s["sem_g"],
                )

        himask = jnp.int32(-65536)

        def half(j, s):
            @pl.when(win(j) < nwin)
            def _():
                w = win(j)
                pltpu.make_async_copy(
                    zs.at[s["ib"].at[pl.ds(0, _WINDOW)]], s["rb"], s["sem_g"]
                ).wait()
                pltpu.make_async_copy(
                    zs.at[s["ib"].at[pl.ds(_WINDOW, _WINDOW)]],
                    s["cb"],
                    s["sem_g"],
                ).wait()

                ib, rb, cb, ob, vb = (
                    s["ib"], s["rb"], s["cb"], s["ob"], s["vb"]
                )

                for e0 in range(0, _WINDOW, _LANES):
                    vb[pl.ds(e0, _LANES)] = plsc.bitcast(
                        ib[pl.ds(2 * _WINDOW + e0, _LANES)], jnp.float32
                    )
                issue_idx(j + _NSLOT, s)

                @pl.when(j >= _NSLOT)
                def _():
                    pltpu.make_async_copy(
                        s["ob"], o_hbm.at[pl.ds(0, _WINDOW)], s["sem_o"]
                    ).wait()

                @plsc.parallel_loop(0, _WINDOW, step=_LANES, unroll=3)
                def _(e0):
                    vvec = vb[pl.ds(e0, _LANES)]
                    for u in range(_LANES):
                        val = vvec[u]
                        for g in range(0, dw, _LANES):
                            r_bf = plsc.bitcast(
                                rb[e0 + u, pl.ds(g, _LANES)], jnp.bfloat16
                            )
                            c_bf = plsc.bitcast(
                                cb[e0 + u, pl.ds(g, _LANES)], jnp.bfloat16
                            )
                            ws = plsc.bitcast(r_bf + c_bf, jnp.int32)
                            ob[e0 + u, pl.ds(g, _LANES)] = val * plsc.bitcast(
                                ws << 16, jnp.float32
                            )
                            ob[e0 + u, pl.ds(dw + g, _LANES)] = (
                                val * plsc.bitcast(ws & himask, jnp.float32)
                            )

                pltpu.async_copy(
                    s["ob"], o_hbm.at[pl.ds(w * _WINDOW, _WINDOW)], s["sem_o"]
                )
                issue_gather(j + _NSLOT, s)

        sub = lax.axis_index("subcore")
        rows_per = n_nodes // 16
        pltpu.sync_copy(
            z_hbm.at[pl.ds(sub * rows_per, rows_per)],
            zs.at[pl.ds(sub * rows_per, rows_per)],
        )
        plsc.subcore_barrier()

        for i in range(_NSLOT):
            issue_idx(i, slots[i])
        for i in range(_NSLOT):
            issue_gather(i, slots[i])

        @pl.loop(0, max_j, step=_NSLOT)
        def _(jj):
            for i in range(_NSLOT):
                half(jj + i, slots[i])

        for s in slots:
            pltpu.make_async_copy(
                o_hbm.at[pl.ds(0, _WINDOW)], s["ob"], s["sem_o"]
            ).wait()

    return k(z_packed, idx_combined)


def kernel(x, adj_rows, adj_cols, adj_vals, W, b):
    n, d_in = x.shape
    d = W.shape[1]
    e_total = adj_rows.shape[0]
    nwin = e_total // _WINDOW
    perm = jnp.arange(d).reshape(2, d // 2).T.reshape(-1)
    z_bf16 = _dense_bf16(x, W[:, perm], b[perm])
    z_packed = lax.bitcast_convert_type(
        z_bf16.reshape(n, d // 2, 2), jnp.int32
    )
    idx_combined = jnp.concatenate(
        [
            adj_rows.reshape(nwin, _WINDOW),
            adj_cols.reshape(nwin, _WINDOW),
            lax.bitcast_convert_type(adj_vals, jnp.int32).reshape(
                nwin, _WINDOW
            ),
        ],
        axis=1,
    )
    return _edge_combine(z_packed, idx_combined, e_total, d)


# speedup vs baseline: 1.7488x; 1.7488x over previous
import dataclasses
import functools

import jax
import jax.numpy as jnp
from jax import lax
from jax.experimental import pallas as pl
from jax.experimental.pallas import tpu as pltpu
from jax.experimental.pallas import tpu_sc as plsc

_LANES = 16
_WINDOW = 128
_NSLOT = 2


def _dense_body(x_ref, w_ref, b_ref, o_ref):
    o_ref[...] = (
        jnp.dot(x_ref[...], w_ref[...], preferred_element_type=jnp.float32)
        + b_ref[...]
    ).astype(jnp.bfloat16)


def _dense_bf16(x, W, b):
    n, d_in = x.shape
    d_out = W.shape[1]
    blk = 2000
    return pl.pallas_call(
        _dense_body,
        grid=(n // blk,),
        in_specs=[
            pl.BlockSpec((blk, d_in), lambda i: (i, 0)),
            pl.BlockSpec((d_in, d_out), lambda i: (0, 0)),
            pl.BlockSpec((1, d_out), lambda i: (0, 0)),
        ],
        out_specs=pl.BlockSpec((blk, d_out), lambda i: (i, 0)),
        out_shape=jax.ShapeDtypeStruct((n, d_out), jnp.bfloat16),
    )(x, W, b.reshape(1, d_out))


def _edge_combine(z_packed, idx_combined, e_total, d):
    nwin = e_total // _WINDOW
    n_nodes = z_packed.shape[0]
    dw = d // 2
    mesh = plsc.VectorSubcoreMesh(
        core_axis_name="core", subcore_axis_name="subcore"
    )
    n_workers = 32
    max_j = -(-nwin // n_workers)
    if max_j % _NSLOT:
        max_j += _NSLOT - max_j % _NSLOT

    cp = pltpu.CompilerParams()
    if "needs_layout_passes" in pltpu.CompilerParams.__dataclass_fields__:
        cp = dataclasses.replace(cp, needs_layout_passes=False)
    if "use_tc_tiling_on_sc" in pltpu.CompilerParams.__dataclass_fields__:
        cp = dataclasses.replace(cp, use_tc_tiling_on_sc=False)

    slot_types = [
        pltpu.VMEM((3 * _WINDOW,), jnp.int32),
        pltpu.VMEM((_WINDOW, dw), jnp.int32),
        pltpu.VMEM((_WINDOW, dw), jnp.int32),
        pltpu.VMEM((_WINDOW, d), jnp.float32),
        pltpu.VMEM((_WINDOW,), jnp.float32),
        pltpu.SemaphoreType.DMA,
        pltpu.SemaphoreType.DMA,
        pltpu.SemaphoreType.DMA,
    ]

    @functools.partial(
        pl.kernel,
        out_type=jax.ShapeDtypeStruct((e_total, d), jnp.float32),
        mesh=mesh,
        compiler_params=cp,
        scratch_types=slot_types * _NSLOT
        + [pltpu.VMEM_SHARED((n_nodes, dw), jnp.int32)],
    )
    def k(z_hbm, i_hbm, o_hbm, *scratch):
        zs = scratch[-1]
        ns = len(slot_types)
        slots = tuple(
            dict(
                ib=scratch[i * ns + 0],
                rb=scratch[i * ns + 1],
                cb=scratch[i * ns + 2],
                ob=scratch[i * ns + 3],
                vb=scratch[i * ns + 4],
                sem_i=scratch[i * ns + 5],
                sem_g=scratch[i * ns + 6],
                sem_o=scratch[i * ns + 7],
            )
            for i in range(_NSLOT)
        )
        t = lax.axis_index("subcore") * 2 + lax.axis_index("core")
        j_per_worker = -(-nwin // n_workers)

        def win(j):
            return t * j_per_worker + j

        def issue_idx(j, s):
            @pl.when(win(j) < nwin)
            def _():
                pltpu.async_copy(i_hbm.at[win(j)], s["ib"], s["sem_i"])

        def issue_gather(j, s):
            @pl.when(win(j) < nwin)
            def _():
                pltpu.make_async_copy(i_hbm.at[0], s["ib"], s["sem_i"]).wait()
                pltpu.async_copy(
                    zs.at[s["ib"].at[pl.ds(0, _WINDOW)]], s["rb"], s["sem_g"]
                )
                pltpu.async_copy(
                    zs.at[s["ib"].at[pl.ds(_WINDOW, _WINDOW)]],
                    s["cb"],
                    s["sem_g"],
                )

        himask = jnp.int32(-65536)

        def half(j, s):
            @pl.when(win(j) < nwin)
            def _():
                w = win(j)
                pltpu.make_async_copy(
                    zs.at[s["ib"].at[pl.ds(0, _WINDOW)]], s["rb"], s["sem_g"]
                ).wait()
                pltpu.make_async_copy(
                    zs.at[s["ib"].at[pl.ds(_WINDOW, _WINDOW)]],
                    s["cb"],
                    s["sem_g"],
                ).wait()

                ib, rb, cb, ob, vb = (
                    s["ib"], s["rb"], s["cb"], s["ob"], s["vb"]
                )

                for e0 in range(0, _WINDOW, _LANES):
                    vb[pl.ds(e0, _LANES)] = plsc.bitcast(
                        ib[pl.ds(2 * _WINDOW + e0, _LANES)], jnp.float32
                    )
                issue_idx(j + _NSLOT, s)

                @pl.when(j >= _NSLOT)
                def _():
                    pltpu.make_async_copy(
                        s["ob"], o_hbm.at[pl.ds(0, _WINDOW)], s["sem_o"]
                    ).wait()

                @plsc.parallel_loop(0, _WINDOW, step=_LANES, unroll=2)
                def _(e0):
                    vvec = vb[pl.ds(e0, _LANES)]
                    for u in range(_LANES):
                        val = vvec[u]
                        for g in range(0, dw, _LANES):
                            r_bf = plsc.bitcast(
                                rb[e0 + u, pl.ds(g, _LANES)], jnp.bfloat16
                            )
                            c_bf = plsc.bitcast(
                                cb[e0 + u, pl.ds(g, _LANES)], jnp.bfloat16
                            )
                            ws = plsc.bitcast(r_bf + c_bf, jnp.int32)
                            ob[e0 + u, pl.ds(g, _LANES)] = val * plsc.bitcast(
                                ws << 16, jnp.float32
                            )
                            ob[e0 + u, pl.ds(dw + g, _LANES)] = (
                                val * plsc.bitcast(ws & himask, jnp.float32)
                            )

                pltpu.async_copy(
                    s["ob"], o_hbm.at[pl.ds(w * _WINDOW, _WINDOW)], s["sem_o"]
                )
                issue_gather(j + _NSLOT, s)

        sub = lax.axis_index("subcore")
        rows_per = n_nodes // 16
        pltpu.sync_copy(
            z_hbm.at[pl.ds(sub * rows_per, rows_per)],
            zs.at[pl.ds(sub * rows_per, rows_per)],
        )
        plsc.subcore_barrier()

        for i in range(_NSLOT):
            issue_idx(i, slots[i])
        for i in range(_NSLOT):
            issue_gather(i, slots[i])

        @pl.loop(0, max_j, step=_NSLOT)
        def _(jj):
            for i in range(_NSLOT):
                half(jj + i, slots[i])

        for s in slots:
            pltpu.make_async_copy(
                o_hbm.at[pl.ds(0, _WINDOW)], s["ob"], s["sem_o"]
            ).wait()

    return k(z_packed, idx_combined)


def kernel(x, adj_rows, adj_cols, adj_vals, W, b):
    n, d_in = x.shape
    d = W.shape[1]
    e_total = adj_rows.shape[0]
    nwin = e_total // _WINDOW
    perm = jnp.arange(d).reshape(2, d // 2).T.reshape(-1)
    z_bf16 = _dense_bf16(x, W[:, perm], b[perm])
    z_packed = lax.bitcast_convert_type(
        z_bf16.reshape(n, d // 2, 2), jnp.int32
    )
    idx_combined = jnp.concatenate(
        [
            adj_rows.reshape(nwin, _WINDOW),
            adj_cols.reshape(nwin, _WINDOW),
            lax.bitcast_convert_type(adj_vals, jnp.int32).reshape(
                nwin, _WINDOW
            ),
        ],
        axis=1,
    )
    return _edge_combine(z_packed, idx_combined, e_total, d)


# speedup vs baseline: 2.0512x; 1.1729x over previous
import dataclasses
import functools

import jax
import jax.numpy as jnp
from jax import lax
from jax.experimental import pallas as pl
from jax.experimental.pallas import tpu as pltpu
from jax.experimental.pallas import tpu_sc as plsc

_LANES = 16
_WINDOW = 128
_NSLOT = 2


def _dense_body(x_ref, w_ref, b_ref, o_ref):
    o_ref[...] = (
        jnp.dot(x_ref[...], w_ref[...], preferred_element_type=jnp.float32)
        + b_ref[...]
    ).astype(jnp.bfloat16)


def _dense_bf16(x, W, b):
    n, d_in = x.shape
    d_out = W.shape[1]
    blk = 2000
    return pl.pallas_call(
        _dense_body,
        grid=(n // blk,),
        in_specs=[
            pl.BlockSpec((blk, d_in), lambda i: (i, 0)),
            pl.BlockSpec((d_in, d_out), lambda i: (0, 0)),
            pl.BlockSpec((1, d_out), lambda i: (0, 0)),
        ],
        out_specs=pl.BlockSpec((blk, d_out), lambda i: (i, 0)),
        out_shape=jax.ShapeDtypeStruct((n, d_out), jnp.bfloat16),
    )(x, W, b.reshape(1, d_out))


def _edge_combine(z_packed, idx_combined, e_total, d):
    nwin = e_total // _WINDOW
    n_nodes = z_packed.shape[0]
    dw = d // 2
    mesh = plsc.VectorSubcoreMesh(
        core_axis_name="core", subcore_axis_name="subcore"
    )
    n_workers = 32
    max_j = -(-nwin // n_workers)
    if max_j % _NSLOT:
        max_j += _NSLOT - max_j % _NSLOT

    cp = pltpu.CompilerParams()
    if "needs_layout_passes" in pltpu.CompilerParams.__dataclass_fields__:
        cp = dataclasses.replace(cp, needs_layout_passes=False)
    if "use_tc_tiling_on_sc" in pltpu.CompilerParams.__dataclass_fields__:
        cp = dataclasses.replace(cp, use_tc_tiling_on_sc=False)

    slot_types = [
        pltpu.VMEM((3 * _WINDOW,), jnp.int32),
        pltpu.VMEM((_WINDOW, dw), jnp.int32),
        pltpu.VMEM((_WINDOW, dw), jnp.int32),
        pltpu.VMEM((_WINDOW, d), jnp.float32),
        pltpu.VMEM((_WINDOW,), jnp.float32),
        pltpu.SemaphoreType.DMA,
        pltpu.SemaphoreType.DMA,
        pltpu.SemaphoreType.DMA,
    ]

    @functools.partial(
        pl.kernel,
        out_type=jax.ShapeDtypeStruct((e_total, d), jnp.float32),
        mesh=mesh,
        compiler_params=cp,
        scratch_types=slot_types * _NSLOT
        + [pltpu.VMEM_SHARED((n_nodes, dw), jnp.int32)],
    )
    def k(z_hbm, i_hbm, o_hbm, *scratch):
        zs = scratch[-1]
        ns = len(slot_types)
        slots = tuple(
            dict(
                ib=scratch[i * ns + 0],
                rb=scratch[i * ns + 1],
                cb=scratch[i * ns + 2],
                ob=scratch[i * ns + 3],
                vb=scratch[i * ns + 4],
                sem_i=scratch[i * ns + 5],
                sem_g=scratch[i * ns + 6],
                sem_o=scratch[i * ns + 7],
            )
            for i in range(_NSLOT)
        )
        t = lax.axis_index("subcore") * 2 + lax.axis_index("core")

        def win(j):
            return t + n_workers * j

        def issue_idx(j, s):
            @pl.when(win(j) < nwin)
            def _():
                pltpu.async_copy(i_hbm.at[win(j)], s["ib"], s["sem_i"])

        def issue_gather(j, s):
            @pl.when(win(j) < nwin)
            def _():
                pltpu.make_async_copy(i_hbm.at[0], s["ib"], s["sem_i"]).wait()
                pltpu.async_copy(
                    zs.at[s["ib"].at[pl.ds(0, _WINDOW)]], s["rb"], s["sem_g"]
                )
                pltpu.async_copy(
                    zs.at[s["ib"].at[pl.ds(_WINDOW, _WINDOW)]],
                    s["cb"],
                    s["sem_g"],
                )

        himask = jnp.int32(-65536)

        def half(j, s):
            @pl.when(win(j) < nwin)
            def _():
                w = win(j)
                pltpu.make_async_copy(
                    zs.at[s["ib"].at[pl.ds(0, _WINDOW)]], s["rb"], s["sem_g"]
                ).wait()
                pltpu.make_async_copy(
                    zs.at[s["ib"].at[pl.ds(_WINDOW, _WINDOW)]],
                    s["cb"],
                    s["sem_g"],
                ).wait()

                ib, rb, cb, ob, vb = (
                    s["ib"], s["rb"], s["cb"], s["ob"], s["vb"]
                )

                for e0 in range(0, _WINDOW, _LANES):
                    vb[pl.ds(e0, _LANES)] = plsc.bitcast(
                        ib[pl.ds(2 * _WINDOW + e0, _LANES)], jnp.float32
                    )
                issue_idx(j + _NSLOT, s)

                @pl.when(j >= _NSLOT)
                def _():
                    pltpu.make_async_copy(
                        s["ob"], o_hbm.at[pl.ds(0, _WINDOW)], s["sem_o"]
                    ).wait()

                for e0 in range(0, _WINDOW, _LANES):
                    vvec = vb[pl.ds(e0, _LANES)]
                    for u in range(_LANES):
                        val = vvec[u]
                        for g in range(0, dw, _LANES):
                            r_bf = plsc.bitcast(
                                rb[e0 + u, pl.ds(g, _LANES)], jnp.bfloat16
                            )
                            c_bf = plsc.bitcast(
                                cb[e0 + u, pl.ds(g, _LANES)], jnp.bfloat16
                            )
                            ws = plsc.bitcast(r_bf + c_bf, jnp.int32)
                            ob[e0 + u, pl.ds(g, _LANES)] = val * plsc.bitcast(
                                ws << 16, jnp.float32
                            )
                            ob[e0 + u, pl.ds(dw + g, _LANES)] = (
                                val * plsc.bitcast(ws & himask, jnp.float32)
                            )

                pltpu.async_copy(
                    s["ob"], o_hbm.at[pl.ds(w * _WINDOW, _WINDOW)], s["sem_o"]
                )
                issue_gather(j + _NSLOT, s)

        sub = lax.axis_index("subcore")
        rows_per = n_nodes // 16
        pltpu.sync_copy(
            z_hbm.at[pl.ds(sub * rows_per, rows_per)],
            zs.at[pl.ds(sub * rows_per, rows_per)],
        )
        plsc.subcore_barrier()

        for i in range(_NSLOT):
            issue_idx(i, slots[i])
        for i in range(_NSLOT):
            issue_gather(i, slots[i])

        @pl.loop(0, max_j, step=_NSLOT)
        def _(jj):
            for i in range(_NSLOT):
                half(jj + i, slots[i])

        for s in slots:
            pltpu.make_async_copy(
                o_hbm.at[pl.ds(0, _WINDOW)], s["ob"], s["sem_o"]
            ).wait()

    return k(z_packed, idx_combined)


def kernel(x, adj_rows, adj_cols, adj_vals, W, b):
    n, d_in = x.shape
    d = W.shape[1]
    e_total = adj_rows.shape[0]
    nwin = e_total // _WINDOW
    perm = jnp.arange(d).reshape(2, d // 2).T.reshape(-1)
    z_bf16 = _dense_bf16(x, W[:, perm], b[perm])
    z_packed = lax.bitcast_convert_type(
        z_bf16.reshape(n, d // 2, 2), jnp.int32
    )
    idx_combined = jnp.concatenate(
        [
            adj_rows.reshape(nwin, _WINDOW),
            adj_cols.reshape(nwin, _WINDOW),
            lax.bitcast_convert_type(adj_vals, jnp.int32).reshape(
                nwin, _WINDOW
            ),
        ],
        axis=1,
    )
    return _edge_combine(z_packed, idx_combined, e_total, d)
